# Initial kernel scaffold; baseline (speedup 1.0000x reference)
import jax
import jax.numpy as jnp
from jax.experimental import pallas as pl


def kernel(x, edge_index, W0_w, W0_b, g1_lin, g1_att_src, g1_att_dst, g1_b, g2_lin, g2_att_src, g2_att_dst, g2_b, bn1_g, bn1_b, bn2_g, bn2_b, Wout_w, Wout_b, bn1_rm, bn1_rv, bn2_rm, bn2_rv):
    raise NotImplementedError("write your pallas kernel here")


# speedup vs baseline: 13.0884x; 13.0884x over previous
import functools
import jax
import jax.numpy as jnp
from jax import lax
from jax.experimental import pallas as pl
from jax.experimental.pallas import tpu as pltpu
from jax.experimental.pallas import tpu_sc as plsc

N = 10000
E = 320000
DIN = 128
H = 4
C = 64
DH = H * C
DOUT = 128
BN_EPS = 1e-5

R = 400
NBLK = N // R

NSC = 2
NTILE = 16
K = 40
EPT = E // NTILE
CHUNKS = EPT // K
DW = 16


def _tc_a_body(x_ref, w0t_ref, b0_ref, g1t_ref, ab_ref, p_ref,
               h0_ref, hs_ref, atabs_ref, atabd_ref, initf_ref, initd_ref):
    xb = x_ref[...]
    h0 = jnp.dot(xb, w0t_ref[...], preferred_element_type=jnp.float32) + b0_ref[...]
    h1 = jnp.dot(h0, g1t_ref[...], preferred_element_type=jnp.float32)
    ab = jnp.dot(h1, ab_ref[...], preferred_element_type=jnp.float32)
    t = ab[:, 0:4] + ab[:, 4:8]
    wself = jnp.exp(jnp.maximum(t, 0.2 * t))
    initf = h1 * jnp.dot(wself, p_ref[...], preferred_element_type=jnp.float32)
    z = jnp.zeros((R, DW - 2), jnp.float32)
    h0_ref[...] = h0
    hs_ref[0] = h1[:, 0:128]
    hs_ref[1] = h1[:, 128:256]
    z12 = jnp.zeros((R, 12), jnp.float32)
    atabs_ref[...] = jnp.concatenate([ab[:, 0:4], z12], axis=1)
    atabd_ref[...] = jnp.concatenate([ab[:, 4:8], z12], axis=1)
    initf_ref[0] = initf[:, 0:128]
    initf_ref[1] = initf[:, 128:256]
    initd_ref[0] = jnp.concatenate([wself[:, 0:2], z], axis=1)
    initd_ref[1] = jnp.concatenate([wself[:, 2:4], z], axis=1)


def _tc_b_body(accf_ref, accd_ref, g1b_ref, s1_ref, sh1_ref, g2t_ref,
               ab2_ref, p_ref,
               out1_ref, hs_ref, atabs_ref, atabd_ref, initf_ref, initd_ref):
    agg = jnp.concatenate([accf_ref[0], accf_ref[1]], axis=1)
    den = jnp.concatenate([accd_ref[0][:, 0:2], accd_ref[1][:, 0:2]], axis=1)
    denr = jnp.dot(den, p_ref[...], preferred_element_type=jnp.float32) + 1e-16
    o = agg / denr + g1b_ref[...]
    o = o * s1_ref[...] + sh1_ref[...]
    out1 = jnp.maximum(o, 0.0)
    h2 = jnp.dot(out1, g2t_ref[...], preferred_element_type=jnp.float32)
    ab = jnp.dot(h2, ab2_ref[...], preferred_element_type=jnp.float32)
    t = ab[:, 0:4] + ab[:, 4:8]
    wself = jnp.exp(jnp.maximum(t, 0.2 * t))
    initf = h2 * jnp.dot(wself, p_ref[...], preferred_element_type=jnp.float32)
    z = jnp.zeros((R, DW - 2), jnp.float32)
    out1_ref[...] = out1
    hs_ref[0] = h2[:, 0:128]
    hs_ref[1] = h2[:, 128:256]
    z12 = jnp.zeros((R, 12), jnp.float32)
    atabs_ref[...] = jnp.concatenate([ab[:, 0:4], z12], axis=1)
    atabd_ref[...] = jnp.concatenate([ab[:, 4:8], z12], axis=1)
    initf_ref[0] = initf[:, 0:128]
    initf_ref[1] = initf[:, 128:256]
    initd_ref[0] = jnp.concatenate([wself[:, 0:2], z], axis=1)
    initd_ref[1] = jnp.concatenate([wself[:, 2:4], z], axis=1)


def _tc_c_body(accf_ref, accd_ref, out1_ref_in, h0_ref_in, g2b_ref,
               s2_ref, sh2_ref, woutt_ref, bout_ref, p_ref, final_ref):
    agg = jnp.concatenate([accf_ref[0], accf_ref[1]], axis=1)
    den = jnp.concatenate([accd_ref[0][:, 0:2], accd_ref[1][:, 0:2]], axis=1)
    denr = jnp.dot(den, p_ref[...], preferred_element_type=jnp.float32) + 1e-16
    o = agg / denr + g2b_ref[...]
    o = o * s2_ref[...] + sh2_ref[...]
    o = o + 0.5 * h0_ref_in[...]
    out2 = jnp.maximum(o, 0.0)
    s = out1_ref_in[...] + out2
    final_ref[...] = jnp.dot(s, woutt_ref[...], preferred_element_type=jnp.float32) + bout_ref[...]


def _full(shape):
    return pl.BlockSpec(shape, lambda i: tuple(0 for _ in shape))


_tc_a = pl.pallas_call(
    _tc_a_body,
    grid=(NBLK,),
    in_specs=[
        pl.BlockSpec((R, DIN), lambda i: (i, 0)),
        _full((DIN, DH)),
        _full((1, DH)),
        _full((DH, DH)),
        _full((DH, 8)),
        _full((H, DH)),
    ],
    out_specs=[
        pl.BlockSpec((R, DH), lambda i: (i, 0)),
        pl.BlockSpec((2, R, 128), lambda i: (0, i, 0)),
        pl.BlockSpec((R, 16), lambda i: (i, 0)),
        pl.BlockSpec((R, 16), lambda i: (i, 0)),
        pl.BlockSpec((2, R, 128), lambda i: (0, i, 0)),
        pl.BlockSpec((2, R, DW), lambda i: (0, i, 0)),
    ],
    out_shape=[
        jax.ShapeDtypeStruct((N, DH), jnp.float32),
        jax.ShapeDtypeStruct((2, N, 128), jnp.float32),
        jax.ShapeDtypeStruct((N, 16), jnp.float32),
        jax.ShapeDtypeStruct((N, 16), jnp.float32),
        jax.ShapeDtypeStruct((2, N, 128), jnp.float32),
        jax.ShapeDtypeStruct((2, N, DW), jnp.float32),
    ],
)

_tc_b = pl.pallas_call(
    _tc_b_body,
    grid=(NBLK,),
    in_specs=[
        pl.BlockSpec((2, R, 128), lambda i: (0, i, 0)),
        pl.BlockSpec((2, R, DW), lambda i: (0, i, 0)),
        _full((1, DH)),
        _full((1, DH)),
        _full((1, DH)),
        _full((DH, DH)),
        _full((DH, 8)),
        _full((H, DH)),
    ],
    out_specs=[
        pl.BlockSpec((R, DH), lambda i: (i, 0)),
        pl.BlockSpec((2, R, 128), lambda i: (0, i, 0)),
        pl.BlockSpec((R, 16), lambda i: (i, 0)),
        pl.BlockSpec((R, 16), lambda i: (i, 0)),
        pl.BlockSpec((2, R, 128), lambda i: (0, i, 0)),
        pl.BlockSpec((2, R, DW), lambda i: (0, i, 0)),
    ],
    out_shape=[
        jax.ShapeDtypeStruct((N, DH), jnp.float32),
        jax.ShapeDtypeStruct((2, N, 128), jnp.float32),
        jax.ShapeDtypeStruct((N, 16), jnp.float32),
        jax.ShapeDtypeStruct((N, 16), jnp.float32),
        jax.ShapeDtypeStruct((2, N, 128), jnp.float32),
        jax.ShapeDtypeStruct((2, N, DW), jnp.float32),
    ],
)

_tc_c = pl.pallas_call(
    _tc_c_body,
    grid=(NBLK,),
    in_specs=[
        pl.BlockSpec((2, R, 128), lambda i: (0, i, 0)),
        pl.BlockSpec((2, R, DW), lambda i: (0, i, 0)),
        pl.BlockSpec((R, DH), lambda i: (i, 0)),
        pl.BlockSpec((R, DH), lambda i: (i, 0)),
        _full((1, DH)),
        _full((1, DH)),
        _full((1, DH)),
        _full((DH, DOUT)),
        _full((1, DOUT)),
        _full((H, DH)),
    ],
    out_specs=[pl.BlockSpec((R, DOUT), lambda i: (i, 0))],
    out_shape=[jax.ShapeDtypeStruct((N, DOUT), jnp.float32)],
)


def _sc_body(eidx_hbm, atabs_hbm, atabd_hbm, hsplit_hbm, initf_hbm, initd_hbm,
             outf_hbm, outd_hbm,
             idx0, ars0, ard0, gath0, sbuf0, dbuf0, stage, dstage,
             accf_sh, accd_sh, gsem):
    c = lax.axis_index("c")
    s = lax.axis_index("s")
    rows_per_tile = 624
    r0 = s * rows_per_tile

    for q in range(8):
        qr = r0 + q * 78
        pltpu.sync_copy(initf_hbm.at[c, pl.ds(qr, 78)], stage)
        pltpu.sync_copy(stage, accf_sh.at[pl.ds(qr, 78)])
    pltpu.sync_copy(initd_hbm.at[c, pl.ds(r0, rows_per_tile)], dstage)
    pltpu.sync_copy(dstage, accd_sh.at[pl.ds(r0, rows_per_tile)])

    @pl.when(s == 0)
    def _():
        tail = NTILE * rows_per_tile
        pltpu.sync_copy(initf_hbm.at[c, pl.ds(tail, 16)],
                        stage.at[pl.ds(0, 16)])
        pltpu.sync_copy(stage.at[pl.ds(0, 16)], accf_sh.at[pl.ds(tail, 16)])
        pltpu.sync_copy(initd_hbm.at[c, pl.ds(tail, 16)],
                        dstage.at[pl.ds(0, 16)])
        pltpu.sync_copy(dstage.at[pl.ds(0, 16)], accd_sh.at[pl.ds(tail, 16)])

    iota = lax.iota(jnp.int32, 16)
    zvec = jnp.zeros((16,), jnp.float32)
    is_core0 = c == 0

    plsc.subcore_barrier()

    def compute():
        def ebody(e, _):
            t = ars0[e, :] + ard0[e, :]
            w = jnp.exp(jnp.maximum(t, 0.2 * t))
            w0e = jnp.where(is_core0, w[0], w[2])
            w1e = jnp.where(is_core0, w[1], w[3])
            w0s = jax.lax.broadcast(w0e, (16,))
            w1s = jax.lax.broadcast(w1e, (16,))
            dbuf0[e, :] = jnp.where(iota == 0, w0s,
                                    jnp.where(iota == 1, w1s, zvec))
            for v in range(8):
                wspl = w0s if v < 4 else w1s
                sbuf0[e, pl.ds(v * 16, 16)] = (
                    gath0[e, pl.ds(v * 16, 16)] * wspl)
            return 0
        lax.fori_loop(0, K, ebody, 0)

    def chunk_body(i, _):
        pltpu.sync_copy(eidx_hbm.at[s].at[i], idx0)
        srcr = idx0.at[0]
        dstr = idx0.at[1]
        pltpu.async_copy(atabs_hbm.at[srcr], ars0, gsem)
        pltpu.async_copy(atabd_hbm.at[dstr], ard0, gsem)
        pltpu.async_copy(hsplit_hbm.at[c].at[srcr], gath0, gsem)
        pltpu.make_async_copy(atabs_hbm.at[srcr], ars0, gsem).wait()
        pltpu.make_async_copy(atabd_hbm.at[dstr], ard0, gsem).wait()
        pltpu.make_async_copy(hsplit_hbm.at[c].at[srcr], gath0, gsem).wait()
        compute()
        pltpu.sync_copy(sbuf0, accf_sh.at[dstr], add=True)
        pltpu.sync_copy(dbuf0, accd_sh.at[dstr], add=True)
        return 0

    lax.fori_loop(0, CHUNKS, chunk_body, 0)

    plsc.subcore_barrier()

    for q in range(8):
        qr = r0 + q * 78
        pltpu.sync_copy(accf_sh.at[pl.ds(qr, 78)], stage)
        pltpu.sync_copy(stage, outf_hbm.at[c, pl.ds(qr, 78)])
    pltpu.sync_copy(accd_sh.at[pl.ds(r0, rows_per_tile)], dstage)
    pltpu.sync_copy(dstage, outd_hbm.at[c, pl.ds(r0, rows_per_tile)])

    @pl.when(s == 0)
    def _():
        tail = NTILE * rows_per_tile
        pltpu.sync_copy(accf_sh.at[pl.ds(tail, 16)], stage.at[pl.ds(0, 16)])
        pltpu.sync_copy(stage.at[pl.ds(0, 16)], outf_hbm.at[c, pl.ds(tail, 16)])
        pltpu.sync_copy(accd_sh.at[pl.ds(tail, 16)], dstage.at[pl.ds(0, 16)])
        pltpu.sync_copy(dstage.at[pl.ds(0, 16)], outd_hbm.at[c, pl.ds(tail, 16)])


@functools.lru_cache(maxsize=1)
def _get_sc_agg():
  return pl.kernel(
    _sc_body,
    out_type=(jax.ShapeDtypeStruct((2, N, 128), jnp.float32),
              jax.ShapeDtypeStruct((2, N, DW), jnp.float32)),
    mesh=plsc.VectorSubcoreMesh(core_axis_name="c", subcore_axis_name="s"),
    compiler_params=pltpu.CompilerParams(
        needs_layout_passes=False, use_tc_tiling_on_sc=False),
    scratch_types=[
        pltpu.VMEM((2, K), jnp.int32),
        pltpu.VMEM((K, 16), jnp.float32),
        pltpu.VMEM((K, 16), jnp.float32),
        pltpu.VMEM((K, 128), jnp.float32),
        pltpu.VMEM((K, 128), jnp.float32),
        pltpu.VMEM((K, DW), jnp.float32),
        pltpu.VMEM((78, 128), jnp.float32),
        pltpu.VMEM((624, DW), jnp.float32),
        pltpu.VMEM_SHARED((N, 128), jnp.float32),
        pltpu.VMEM_SHARED((N, DW), jnp.float32),
        pltpu.SemaphoreType.DMA,
    ],
  )


def _att_mat(att_s, att_d):
    rows = jnp.arange(DH) // C
    onehot = (rows[:, None] == jnp.arange(H)[None, :]).astype(jnp.float32)
    As = onehot * jnp.reshape(att_s, (DH,))[:, None]
    Ad = onehot * jnp.reshape(att_d, (DH,))[:, None]
    return jnp.concatenate([As, Ad], axis=1)


@jax.jit
def kernel(x, edge_index, W0_w, W0_b, g1_lin, g1_att_src, g1_att_dst, g1_b,
           g2_lin, g2_att_src, g2_att_dst, g2_b, bn1_g, bn1_b, bn2_g, bn2_b,
           Wout_w, Wout_b, bn1_rm, bn1_rv, bn2_rm, bn2_rv):
    eidx = jnp.stack([edge_index[0].astype(jnp.int32).reshape(NTILE, CHUNKS, K),
                      edge_index[1].astype(jnp.int32).reshape(NTILE, CHUNKS, K)],
                     axis=2)

    P = jnp.kron(jnp.eye(H, dtype=jnp.float32), jnp.ones((1, C), jnp.float32))
    AB1 = _att_mat(g1_att_src, g1_att_dst)
    AB2 = _att_mat(g2_att_src, g2_att_dst)
    s1 = bn1_g / jnp.sqrt(bn1_rv + BN_EPS)
    sh1 = bn1_b - bn1_rm * s1
    s2 = bn2_g / jnp.sqrt(bn2_rv + BN_EPS)
    sh2 = bn2_b - bn2_rm * s2

    r2 = lambda v: v.reshape(1, -1)

    sc_agg = _get_sc_agg()
    h0, hs1, atabs1, atabd1, initf1, initd1 = _tc_a(
        x, W0_w.T, r2(W0_b), g1_lin.T, AB1, P)
    accf1, accd1 = sc_agg(eidx, atabs1, atabd1, hs1, initf1, initd1)
    out1, hs2, atabs2, atabd2, initf2, initd2 = _tc_b(
        accf1, accd1, r2(g1_b), r2(s1), r2(sh1), g2_lin.T, AB2, P)
    accf2, accd2 = sc_agg(eidx, atabs2, atabd2, hs2, initf2, initd2)
    (final,) = _tc_c(
        accf2, accd2, out1, h0, r2(g2_b), r2(s2), r2(sh2), Wout_w.T,
        r2(Wout_b), P)
    return final


# speedup vs baseline: 16.4566x; 1.2573x over previous
import functools
import jax
import jax.numpy as jnp
from jax import lax
from jax.experimental import pallas as pl
from jax.experimental.pallas import tpu as pltpu
from jax.experimental.pallas import tpu_sc as plsc

N = 10000
E = 320000
DIN = 128
H = 4
C = 64
DH = H * C
DOUT = 128
BN_EPS = 1e-5

R = 400
NBLK = N // R

NSC = 2
NTILE = 16
K = 40
EPT = E // NTILE
CHUNKS = EPT // K
DW = 16


def _tc_a_body(x_ref, w0t_ref, b0_ref, g1t_ref, ab_ref, p_ref,
               h0_ref, hs_ref, atabs_ref, atabd_ref, initf_ref, initd_ref):
    xb = x_ref[...]
    h0 = jnp.dot(xb, w0t_ref[...], preferred_element_type=jnp.float32) + b0_ref[...]
    h1 = jnp.dot(h0, g1t_ref[...], preferred_element_type=jnp.float32)
    ab = jnp.dot(h1, ab_ref[...], preferred_element_type=jnp.float32)
    t = ab[:, 0:4] + ab[:, 4:8]
    wself = jnp.exp(jnp.maximum(t, 0.2 * t))
    initf = h1 * jnp.dot(wself, p_ref[...], preferred_element_type=jnp.float32)
    z = jnp.zeros((R, DW - 2), jnp.float32)
    h0_ref[...] = h0
    hs_ref[0] = h1[:, 0:128]
    hs_ref[1] = h1[:, 128:256]
    z12 = jnp.zeros((R, 12), jnp.float32)
    atabs_ref[...] = jnp.concatenate([ab[:, 0:4], z12], axis=1)
    atabd_ref[...] = jnp.concatenate([ab[:, 4:8], z12], axis=1)
    initf_ref[0] = initf[:, 0:128]
    initf_ref[1] = initf[:, 128:256]
    initd_ref[0] = jnp.concatenate([wself[:, 0:2], z], axis=1)
    initd_ref[1] = jnp.concatenate([wself[:, 2:4], z], axis=1)


def _tc_b_body(accf_ref, accd_ref, g1b_ref, s1_ref, sh1_ref, g2t_ref,
               ab2_ref, p_ref,
               out1_ref, hs_ref, atabs_ref, atabd_ref, initf_ref, initd_ref):
    agg = jnp.concatenate([accf_ref[0], accf_ref[1]], axis=1)
    den = jnp.concatenate([accd_ref[0][:, 0:2], accd_ref[1][:, 0:2]], axis=1)
    denr = jnp.dot(den, p_ref[...], preferred_element_type=jnp.float32) + 1e-16
    o = agg / denr + g1b_ref[...]
    o = o * s1_ref[...] + sh1_ref[...]
    out1 = jnp.maximum(o, 0.0)
    h2 = jnp.dot(out1, g2t_ref[...], preferred_element_type=jnp.float32)
    ab = jnp.dot(h2, ab2_ref[...], preferred_element_type=jnp.float32)
    t = ab[:, 0:4] + ab[:, 4:8]
    wself = jnp.exp(jnp.maximum(t, 0.2 * t))
    initf = h2 * jnp.dot(wself, p_ref[...], preferred_element_type=jnp.float32)
    z = jnp.zeros((R, DW - 2), jnp.float32)
    out1_ref[...] = out1
    hs_ref[0] = h2[:, 0:128]
    hs_ref[1] = h2[:, 128:256]
    z12 = jnp.zeros((R, 12), jnp.float32)
    atabs_ref[...] = jnp.concatenate([ab[:, 0:4], z12], axis=1)
    atabd_ref[...] = jnp.concatenate([ab[:, 4:8], z12], axis=1)
    initf_ref[0] = initf[:, 0:128]
    initf_ref[1] = initf[:, 128:256]
    initd_ref[0] = jnp.concatenate([wself[:, 0:2], z], axis=1)
    initd_ref[1] = jnp.concatenate([wself[:, 2:4], z], axis=1)


def _tc_c_body(accf_ref, accd_ref, out1_ref_in, h0_ref_in, g2b_ref,
               s2_ref, sh2_ref, woutt_ref, bout_ref, p_ref, final_ref):
    agg = jnp.concatenate([accf_ref[0], accf_ref[1]], axis=1)
    den = jnp.concatenate([accd_ref[0][:, 0:2], accd_ref[1][:, 0:2]], axis=1)
    denr = jnp.dot(den, p_ref[...], preferred_element_type=jnp.float32) + 1e-16
    o = agg / denr + g2b_ref[...]
    o = o * s2_ref[...] + sh2_ref[...]
    o = o + 0.5 * h0_ref_in[...]
    out2 = jnp.maximum(o, 0.0)
    s = out1_ref_in[...] + out2
    final_ref[...] = jnp.dot(s, woutt_ref[...], preferred_element_type=jnp.float32) + bout_ref[...]


def _full(shape):
    return pl.BlockSpec(shape, lambda i: tuple(0 for _ in shape))


_tc_a = pl.pallas_call(
    _tc_a_body,
    grid=(NBLK,),
    in_specs=[
        pl.BlockSpec((R, DIN), lambda i: (i, 0)),
        _full((DIN, DH)),
        _full((1, DH)),
        _full((DH, DH)),
        _full((DH, 8)),
        _full((H, DH)),
    ],
    out_specs=[
        pl.BlockSpec((R, DH), lambda i: (i, 0)),
        pl.BlockSpec((2, R, 128), lambda i: (0, i, 0)),
        pl.BlockSpec((R, 16), lambda i: (i, 0)),
        pl.BlockSpec((R, 16), lambda i: (i, 0)),
        pl.BlockSpec((2, R, 128), lambda i: (0, i, 0)),
        pl.BlockSpec((2, R, DW), lambda i: (0, i, 0)),
    ],
    out_shape=[
        jax.ShapeDtypeStruct((N, DH), jnp.float32),
        jax.ShapeDtypeStruct((2, N, 128), jnp.float32),
        jax.ShapeDtypeStruct((N, 16), jnp.float32),
        jax.ShapeDtypeStruct((N, 16), jnp.float32),
        jax.ShapeDtypeStruct((2, N, 128), jnp.float32),
        jax.ShapeDtypeStruct((2, N, DW), jnp.float32),
    ],
)

_tc_b = pl.pallas_call(
    _tc_b_body,
    grid=(NBLK,),
    in_specs=[
        pl.BlockSpec((2, R, 128), lambda i: (0, i, 0)),
        pl.BlockSpec((2, R, DW), lambda i: (0, i, 0)),
        _full((1, DH)),
        _full((1, DH)),
        _full((1, DH)),
        _full((DH, DH)),
        _full((DH, 8)),
        _full((H, DH)),
    ],
    out_specs=[
        pl.BlockSpec((R, DH), lambda i: (i, 0)),
        pl.BlockSpec((2, R, 128), lambda i: (0, i, 0)),
        pl.BlockSpec((R, 16), lambda i: (i, 0)),
        pl.BlockSpec((R, 16), lambda i: (i, 0)),
        pl.BlockSpec((2, R, 128), lambda i: (0, i, 0)),
        pl.BlockSpec((2, R, DW), lambda i: (0, i, 0)),
    ],
    out_shape=[
        jax.ShapeDtypeStruct((N, DH), jnp.float32),
        jax.ShapeDtypeStruct((2, N, 128), jnp.float32),
        jax.ShapeDtypeStruct((N, 16), jnp.float32),
        jax.ShapeDtypeStruct((N, 16), jnp.float32),
        jax.ShapeDtypeStruct((2, N, 128), jnp.float32),
        jax.ShapeDtypeStruct((2, N, DW), jnp.float32),
    ],
)

_tc_c = pl.pallas_call(
    _tc_c_body,
    grid=(NBLK,),
    in_specs=[
        pl.BlockSpec((2, R, 128), lambda i: (0, i, 0)),
        pl.BlockSpec((2, R, DW), lambda i: (0, i, 0)),
        pl.BlockSpec((R, DH), lambda i: (i, 0)),
        pl.BlockSpec((R, DH), lambda i: (i, 0)),
        _full((1, DH)),
        _full((1, DH)),
        _full((1, DH)),
        _full((DH, DOUT)),
        _full((1, DOUT)),
        _full((H, DH)),
    ],
    out_specs=[pl.BlockSpec((R, DOUT), lambda i: (i, 0))],
    out_shape=[jax.ShapeDtypeStruct((N, DOUT), jnp.float32)],
)


def _sc_body(eidx_hbm, atabs_hbm, atabd_hbm, hsplit_hbm, initf_hbm, initd_hbm,
             outf_hbm, outd_hbm,
             idx0, idx1, ars0, ars1, ard0, ard1, gath0, gath1,
             sbuf0, dbuf0, stage, dstage,
             accf_sh, accd_sh, gsem0, gsem1):
    idx = (idx0, idx1)
    ars = (ars0, ars1)
    ard = (ard0, ard1)
    gath = (gath0, gath1)
    gsem = (gsem0, gsem1)
    c = lax.axis_index("c")
    s = lax.axis_index("s")
    rows_per_tile = 624
    r0 = s * rows_per_tile

    for q in range(8):
        qr = r0 + q * 78
        pltpu.sync_copy(initf_hbm.at[c, pl.ds(qr, 78)], stage)
        pltpu.sync_copy(stage, accf_sh.at[pl.ds(qr, 78)])
    pltpu.sync_copy(initd_hbm.at[c, pl.ds(r0, rows_per_tile)], dstage)
    pltpu.sync_copy(dstage, accd_sh.at[pl.ds(r0, rows_per_tile)])

    @pl.when(s == 0)
    def _():
        tail = NTILE * rows_per_tile
        pltpu.sync_copy(initf_hbm.at[c, pl.ds(tail, 16)],
                        stage.at[pl.ds(0, 16)])
        pltpu.sync_copy(stage.at[pl.ds(0, 16)], accf_sh.at[pl.ds(tail, 16)])
        pltpu.sync_copy(initd_hbm.at[c, pl.ds(tail, 16)],
                        dstage.at[pl.ds(0, 16)])
        pltpu.sync_copy(dstage.at[pl.ds(0, 16)], accd_sh.at[pl.ds(tail, 16)])

    iota = lax.iota(jnp.int32, 16)
    zvec = jnp.zeros((16,), jnp.float32)
    is_core0 = c == 0

    plsc.subcore_barrier()

    def compute(b):
        def ebody(e, _):
            t = ars[b][e, :] + ard[b][e, :]
            w = jnp.exp(jnp.maximum(t, 0.2 * t))
            w0e = jnp.where(is_core0, w[0], w[2])
            w1e = jnp.where(is_core0, w[1], w[3])
            w0s = jax.lax.broadcast(w0e, (16,))
            w1s = jax.lax.broadcast(w1e, (16,))
            dbuf0[e, :] = jnp.where(iota == 0, w0s,
                                    jnp.where(iota == 1, w1s, zvec))
            for v in range(8):
                wspl = w0s if v < 4 else w1s
                sbuf0[e, pl.ds(v * 16, 16)] = (
                    gath[b][e, pl.ds(v * 16, 16)] * wspl)
            return 0
        lax.fori_loop(0, K, ebody, 0)

    def fetch(i, b):
        pltpu.sync_copy(eidx_hbm.at[s].at[i], idx[b])
        srcr = idx[b].at[0]
        dstr = idx[b].at[1]
        pltpu.async_copy(atabs_hbm.at[srcr], ars[b], gsem[b])
        pltpu.async_copy(atabd_hbm.at[dstr], ard[b], gsem[b])
        pltpu.async_copy(hsplit_hbm.at[c].at[srcr], gath[b], gsem[b])

    def finish(b):
        srcr = idx[b].at[0]
        dstr = idx[b].at[1]
        pltpu.make_async_copy(atabs_hbm.at[srcr], ars[b], gsem[b]).wait()
        pltpu.make_async_copy(atabd_hbm.at[dstr], ard[b], gsem[b]).wait()
        pltpu.make_async_copy(hsplit_hbm.at[c].at[srcr], gath[b], gsem[b]).wait()
        compute(b)
        pltpu.sync_copy(sbuf0, accf_sh.at[dstr], add=True)
        pltpu.sync_copy(dbuf0, accd_sh.at[dstr], add=True)

    fetch(0, 0)

    def pair_body(p, _):
        i0 = 2 * p
        fetch(i0 + 1, 1)
        finish(0)

        @pl.when(p < CHUNKS // 2 - 1)
        def _():
            fetch(i0 + 2, 0)
        finish(1)
        return 0

    lax.fori_loop(0, CHUNKS // 2, pair_body, 0)

    plsc.subcore_barrier()

    for q in range(8):
        qr = r0 + q * 78
        pltpu.sync_copy(accf_sh.at[pl.ds(qr, 78)], stage)
        pltpu.sync_copy(stage, outf_hbm.at[c, pl.ds(qr, 78)])
    pltpu.sync_copy(accd_sh.at[pl.ds(r0, rows_per_tile)], dstage)
    pltpu.sync_copy(dstage, outd_hbm.at[c, pl.ds(r0, rows_per_tile)])

    @pl.when(s == 0)
    def _():
        tail = NTILE * rows_per_tile
        pltpu.sync_copy(accf_sh.at[pl.ds(tail, 16)], stage.at[pl.ds(0, 16)])
        pltpu.sync_copy(stage.at[pl.ds(0, 16)], outf_hbm.at[c, pl.ds(tail, 16)])
        pltpu.sync_copy(accd_sh.at[pl.ds(tail, 16)], dstage.at[pl.ds(0, 16)])
        pltpu.sync_copy(dstage.at[pl.ds(0, 16)], outd_hbm.at[c, pl.ds(tail, 16)])


@functools.lru_cache(maxsize=1)
def _get_sc_agg():
  return pl.kernel(
    _sc_body,
    out_type=(jax.ShapeDtypeStruct((2, N, 128), jnp.float32),
              jax.ShapeDtypeStruct((2, N, DW), jnp.float32)),
    mesh=plsc.VectorSubcoreMesh(core_axis_name="c", subcore_axis_name="s"),
    compiler_params=pltpu.CompilerParams(
        needs_layout_passes=False, use_tc_tiling_on_sc=False),
    scratch_types=[
        pltpu.VMEM((2, K), jnp.int32),
        pltpu.VMEM((2, K), jnp.int32),
        pltpu.VMEM((K, 16), jnp.float32),
        pltpu.VMEM((K, 16), jnp.float32),
        pltpu.VMEM((K, 16), jnp.float32),
        pltpu.VMEM((K, 16), jnp.float32),
        pltpu.VMEM((K, 128), jnp.float32),
        pltpu.VMEM((K, 128), jnp.float32),
        pltpu.VMEM((K, 128), jnp.float32),
        pltpu.VMEM((K, DW), jnp.float32),
        pltpu.VMEM((78, 128), jnp.float32),
        pltpu.VMEM((624, DW), jnp.float32),
        pltpu.VMEM_SHARED((N, 128), jnp.float32),
        pltpu.VMEM_SHARED((N, DW), jnp.float32),
        pltpu.SemaphoreType.DMA,
        pltpu.SemaphoreType.DMA,
    ],
  )


def _att_mat(att_s, att_d):
    rows = jnp.arange(DH) // C
    onehot = (rows[:, None] == jnp.arange(H)[None, :]).astype(jnp.float32)
    As = onehot * jnp.reshape(att_s, (DH,))[:, None]
    Ad = onehot * jnp.reshape(att_d, (DH,))[:, None]
    return jnp.concatenate([As, Ad], axis=1)


@jax.jit
def kernel(x, edge_index, W0_w, W0_b, g1_lin, g1_att_src, g1_att_dst, g1_b,
           g2_lin, g2_att_src, g2_att_dst, g2_b, bn1_g, bn1_b, bn2_g, bn2_b,
           Wout_w, Wout_b, bn1_rm, bn1_rv, bn2_rm, bn2_rv):
    eidx = jnp.stack([edge_index[0].astype(jnp.int32).reshape(NTILE, CHUNKS, K),
                      edge_index[1].astype(jnp.int32).reshape(NTILE, CHUNKS, K)],
                     axis=2)

    P = jnp.kron(jnp.eye(H, dtype=jnp.float32), jnp.ones((1, C), jnp.float32))
    AB1 = _att_mat(g1_att_src, g1_att_dst)
    AB2 = _att_mat(g2_att_src, g2_att_dst)
    s1 = bn1_g / jnp.sqrt(bn1_rv + BN_EPS)
    sh1 = bn1_b - bn1_rm * s1
    s2 = bn2_g / jnp.sqrt(bn2_rv + BN_EPS)
    sh2 = bn2_b - bn2_rm * s2

    r2 = lambda v: v.reshape(1, -1)

    sc_agg = _get_sc_agg()
    h0, hs1, atabs1, atabd1, initf1, initd1 = _tc_a(
        x, W0_w.T, r2(W0_b), g1_lin.T, AB1, P)
    accf1, accd1 = sc_agg(eidx, atabs1, atabd1, hs1, initf1, initd1)
    out1, hs2, atabs2, atabd2, initf2, initd2 = _tc_b(
        accf1, accd1, r2(g1_b), r2(s1), r2(sh1), g2_lin.T, AB2, P)
    accf2, accd2 = sc_agg(eidx, atabs2, atabd2, hs2, initf2, initd2)
    (final,) = _tc_c(
        accf2, accd2, out1, h0, r2(g2_b), r2(s2), r2(sh2), Wout_w.T,
        r2(Wout_b), P)
    return final
